# Initial kernel scaffold; baseline (speedup 1.0000x reference)
import jax
import jax.numpy as jnp
from jax.experimental import pallas as pl


def kernel(x, flow):
    raise NotImplementedError("write your pallas kernel here")


# speedup vs baseline: 6.5034x; 6.5034x over previous
import jax
import jax.numpy as jnp
from jax import lax
from jax.experimental import pallas as pl
from jax.experimental.pallas import tpu as pltpu
from jax.experimental.pallas import tpu_sc as plsc

H = 384
W = 384
HW = H * W
NFRAMES = 16
CHUNK_ROWS = 32
CHUNK = CHUNK_ROWS * W
NCHUNK = H // CHUNK_ROWS
GROUPS_PER_ROW = W // 16
MAGIC = 8388608.0


def _fire_body(x_hbm, flow_hbm, out_hbm, fx_v, fy_v, idx_v, g_v, xc_v, sem):
    cid = lax.axis_index("c")
    sid = lax.axis_index("s")
    wid = sid * 2 + cid
    tm1 = wid // 2
    d = wid % 2
    f = jnp.where(d == 0, tm1, 31 - tm1)
    src_t = jnp.where(d == 0, tm1 + 2, tm1)
    cur_t = tm1 + 1
    out_frame = tm1 * 6 + d * 3

    iota_f = lax.iota(jnp.int32, 16).astype(jnp.float32)

    def chunk_body(c, _):
        base = c * CHUNK
        pltpu.sync_copy(flow_hbm.at[pl.ds(2 * f * HW + base, CHUNK)], fx_v)
        pltpu.sync_copy(flow_hbm.at[pl.ds((2 * f + 1) * HW + base, CHUNK)], fy_v)

        row0_f = (c * CHUNK_ROWS).astype(jnp.float32)

        def row_body(r, _):
            row_f = row0_f + r.astype(jnp.float32)
            roff = r * W
            for k in range(GROUPS_PER_ROW):
                off = roff + k * 16
                fx = fx_v[pl.ds(off, 16)]
                fy = fy_v[pl.ds(off, 16)]
                rx = jnp.minimum(jnp.maximum(fx + row_f, 0.0), float(H - 1))
                rx = (rx + MAGIC) - MAGIC
                colv = iota_f + float(k * 16)
                ry = jnp.minimum(jnp.maximum(fy + colv, 0.0), float(W - 1))
                ry = (ry + MAGIC) - MAGIC
                flat = rx * float(W) + ry
                idx_v[pl.ds(off, 16)] = flat.astype(jnp.int32)
            return 0

        lax.fori_loop(0, CHUNK_ROWS, row_body, 0)

        for ch in range(3):
            src_base = (src_t * 3 + ch) * HW
            pltpu.async_copy(
                x_hbm.at[pl.ds(src_base, HW)].at[idx_v], g_v, sem
            ).wait()
            pltpu.sync_copy(
                x_hbm.at[pl.ds((cur_t * 3 + ch) * HW + base, CHUNK)], xc_v
            )

            def sub_body(gg, _):
                off = gg * 64
                for u in range(4):
                    o = off + u * 16
                    g_v[pl.ds(o, 16)] = xc_v[pl.ds(o, 16)] - g_v[pl.ds(o, 16)]
                return 0

            lax.fori_loop(0, CHUNK // 64, sub_body, 0)
            pltpu.sync_copy(
                g_v, out_hbm.at[pl.ds((out_frame + ch) * HW + base, CHUNK)]
            )
        return 0

    lax.fori_loop(0, NCHUNK, chunk_body, 0)


@jax.jit
def kernel(x, flow):
    x_flat = x.reshape(-1)
    flow_flat = flow.reshape(-1)

    mesh = plsc.VectorSubcoreMesh(core_axis_name="c", subcore_axis_name="s")
    out = pl.kernel(
        _fire_body,
        out_type=jax.ShapeDtypeStruct((NFRAMES * 6 * HW,), jnp.float32),
        mesh=mesh,
        scratch_types=[
            pltpu.VMEM((CHUNK,), jnp.float32),
            pltpu.VMEM((CHUNK,), jnp.float32),
            pltpu.VMEM((CHUNK,), jnp.int32),
            pltpu.VMEM((CHUNK,), jnp.float32),
            pltpu.VMEM((CHUNK,), jnp.float32),
            pltpu.SemaphoreType.DMA,
        ],
    )(x_flat, flow_flat)
    return out.reshape(NFRAMES, 6, H, W)


# speedup vs baseline: 7.2142x; 1.1093x over previous
import jax
import jax.numpy as jnp
from jax import lax
from jax.experimental import pallas as pl
from jax.experimental.pallas import tpu as pltpu
from jax.experimental.pallas import tpu_sc as plsc

H = 384
W = 384
HW = H * W
NFRAMES = 16
CHUNK_ROWS = 24
CHUNK = CHUNK_ROWS * W
NCHUNK = H // CHUNK_ROWS
GROUPS_PER_ROW = W // 16
MAGIC = 8388608.0


def _fire_body(x_hbm, flow_hbm, out_hbm, fx_v, fy_v, idx0, idx1,
               g00, g01, g02, g10, g11, g12, xc0, xc1, xc2,
               sem_g, sem_xc, sem_f, sem_st0, sem_st1):
    idx_b = (idx0, idx1)
    g_b = ((g00, g01, g02), (g10, g11, g12))
    xc_b = (xc0, xc1, xc2)

    cid = lax.axis_index("c")
    sid = lax.axis_index("s")
    wid = sid * 2 + cid
    tm1 = wid // 2
    d = wid % 2
    f = jnp.where(d == 0, tm1, 31 - tm1)
    src_t = jnp.where(d == 0, tm1 + 2, tm1)
    cur_t = tm1 + 1
    out_frame = tm1 * 6 + d * 3

    iota_f = lax.iota(jnp.int32, 16).astype(jnp.float32)

    def compute_idx(c, pp):
        row0_f = (c * CHUNK_ROWS).astype(jnp.float32)
        idx_v = idx_b[pp]

        def row_body(r, _):
            row_f = row0_f + r.astype(jnp.float32)
            roff = r * W
            for k in range(GROUPS_PER_ROW):
                off = roff + k * 16
                fx = fx_v[pl.ds(off, 16)]
                fy = fy_v[pl.ds(off, 16)]
                rx = jnp.minimum(jnp.maximum(fx + row_f, 0.0), float(H - 1))
                rx = (rx + MAGIC) - MAGIC
                colv = iota_f + float(k * 16)
                ry = jnp.minimum(jnp.maximum(fy + colv, 0.0), float(W - 1))
                ry = (ry + MAGIC) - MAGIC
                flat = rx * float(W) + ry
                idx_v[pl.ds(off, 16)] = flat.astype(jnp.int32)
            return 0

        lax.fori_loop(0, CHUNK_ROWS, row_body, 0)

    def load_flow(c):
        base = c * CHUNK
        pltpu.async_copy(flow_hbm.at[pl.ds(2 * f * HW + base, CHUNK)], fx_v, sem_f)
        pltpu.async_copy(
            flow_hbm.at[pl.ds((2 * f + 1) * HW + base, CHUNK)], fy_v, sem_f
        )

    def wait_flow():
        pltpu.make_async_copy(flow_hbm.at[pl.ds(0, CHUNK)], fx_v, sem_f).wait()
        pltpu.make_async_copy(flow_hbm.at[pl.ds(0, CHUNK)], fy_v, sem_f).wait()

    load_flow(0)
    wait_flow()
    compute_idx(jnp.int32(0), 0)

    def do_chunk(s, c, pp, sem_st, first, last):
        base = c * CHUNK
        g_p = g_b[pp]

        if not first:

            @pl.when(s >= 1)
            def _():
                for ch in range(3):
                    pltpu.make_async_copy(
                        g_p[ch],
                        out_hbm.at[pl.ds((out_frame + ch) * HW + base, CHUNK)],
                        sem_st,
                    ).wait()

        for ch in range(3):
            src_base = (src_t * 3 + ch) * HW
            pltpu.async_copy(
                x_hbm.at[pl.ds(src_base, HW)].at[idx_b[pp]], g_p[ch], sem_g
            )
        for ch in range(3):
            pltpu.async_copy(
                x_hbm.at[pl.ds((cur_t * 3 + ch) * HW + base, CHUNK)],
                xc_b[ch],
                sem_xc,
            )

        if not last:
            load_flow(c + 1)
            wait_flow()
            compute_idx(c + 1, (pp + 1) % 2)

        for ch in range(3):
            pltpu.make_async_copy(
                x_hbm.at[pl.ds(0, HW)].at[idx_b[pp]], g_p[ch], sem_g
            ).wait()
            pltpu.make_async_copy(
                x_hbm.at[pl.ds(0, CHUNK)], xc_b[ch], sem_xc
            ).wait()

        def sub_body(gg, _):
            off = gg * 64
            for ch in range(3):
                for u in range(4):
                    o = off + u * 16
                    g_p[ch][pl.ds(o, 16)] = (
                        xc_b[ch][pl.ds(o, 16)] - g_p[ch][pl.ds(o, 16)]
                    )
            return 0

        lax.fori_loop(0, CHUNK // 64, sub_body, 0)

        for ch in range(3):
            pltpu.async_copy(
                g_p[ch],
                out_hbm.at[pl.ds((out_frame + ch) * HW + base, CHUNK)],
                sem_st,
            )

    def super_body(s, _):
        do_chunk(s, 2 * s, 0, sem_st0, first=False, last=False)
        do_chunk(s, 2 * s + 1, 1, sem_st1, first=False, last=False)
        return 0

    do_chunk(jnp.int32(0), jnp.int32(0), 0, sem_st0, first=True, last=False)
    do_chunk(jnp.int32(0), jnp.int32(1), 1, sem_st1, first=False, last=False)
    lax.fori_loop(1, NCHUNK // 2 - 1, super_body, 0)
    s_last = jnp.int32(NCHUNK // 2 - 1)
    do_chunk(s_last, 2 * s_last, 0, sem_st0, first=False, last=False)
    do_chunk(s_last, 2 * s_last + 1, 1, sem_st1, first=False, last=True)

    for pp, sem_st in ((0, sem_st0), (1, sem_st1)):
        base = (NCHUNK - 2 + pp) * CHUNK
        for ch in range(3):
            pltpu.make_async_copy(
                g_b[pp][ch],
                out_hbm.at[pl.ds((out_frame + ch) * HW + base, CHUNK)],
                sem_st,
            ).wait()


@jax.jit
def kernel(x, flow):
    x_flat = x.reshape(-1)
    flow_flat = flow.reshape(-1)

    mesh = plsc.VectorSubcoreMesh(core_axis_name="c", subcore_axis_name="s")
    out = pl.kernel(
        _fire_body,
        out_type=jax.ShapeDtypeStruct((NFRAMES * 6 * HW,), jnp.float32),
        mesh=mesh,
        scratch_types=[
            pltpu.VMEM((CHUNK,), jnp.float32),
            pltpu.VMEM((CHUNK,), jnp.float32),
            pltpu.VMEM((CHUNK,), jnp.int32),
            pltpu.VMEM((CHUNK,), jnp.int32),
            pltpu.VMEM((CHUNK,), jnp.float32),
            pltpu.VMEM((CHUNK,), jnp.float32),
            pltpu.VMEM((CHUNK,), jnp.float32),
            pltpu.VMEM((CHUNK,), jnp.float32),
            pltpu.VMEM((CHUNK,), jnp.float32),
            pltpu.VMEM((CHUNK,), jnp.float32),
            pltpu.VMEM((CHUNK,), jnp.float32),
            pltpu.VMEM((CHUNK,), jnp.float32),
            pltpu.VMEM((CHUNK,), jnp.float32),
            pltpu.SemaphoreType.DMA,
            pltpu.SemaphoreType.DMA,
            pltpu.SemaphoreType.DMA,
            pltpu.SemaphoreType.DMA,
            pltpu.SemaphoreType.DMA,
        ],
    )(x_flat, flow_flat)
    return out.reshape(NFRAMES, 6, H, W)
